# baseline (speedup 1.0000x reference)
import math

import jax
import jax.numpy as jnp
from jax import lax
from jax.experimental import pallas as pl
from jax.experimental.pallas import tpu as pltpu
from jax.experimental.pallas import tpu_sc as plsc

EMB = 64
SCALE = math.sqrt(EMB)
ROWS = 4096
SEQ = 50
NC, NS, L = 2, 16, 16
NW = NC * NS
R_PER_W = ROWS // NW
RCH = 16
NCH = R_PER_W // RCH


def _emb_body(tok_hbm, table_hbm, out_hbm, idx_v, buf, gsem):
    wid = lax.axis_index("s") * NC + lax.axis_index("c")
    base = wid * R_PER_W
    pltpu.sync_copy(tok_hbm.at[pl.ds(base, R_PER_W)], idx_v)

    def chunk_body(g, carry):
        cps = [
            pltpu.async_copy(
                table_hbm.at[idx_v.at[g * RCH + r]],
                buf.at[r],
                gsem,
            )
            for r in range(RCH)
        ]
        for cp in cps:
            cp.wait()

        def mul_row(i, c):
            def mul_tok(j, c2):
                for l in range(EMB // L):
                    buf[i, j, pl.ds(l * L, L)] = buf[i, j, pl.ds(l * L, L)] * SCALE
                return c2

            return lax.fori_loop(0, SEQ, mul_tok, c)

        lax.fori_loop(0, RCH, mul_row, 0)

        pltpu.sync_copy(buf, out_hbm.at[pl.ds(base + g * RCH, RCH)])
        return carry

    lax.fori_loop(0, NCH, chunk_body, 0)


@jax.jit
def _emb_call(tokens, table):
    mesh = plsc.VectorSubcoreMesh(core_axis_name="c", subcore_axis_name="s")
    return pl.kernel(
        _emb_body,
        mesh=mesh,
        compiler_params=pltpu.CompilerParams(use_tc_tiling_on_sc=False),
        out_type=jax.ShapeDtypeStruct((ROWS, SEQ, EMB), jnp.float32),
        scratch_types=[
            pltpu.VMEM((R_PER_W, SEQ), jnp.int32),
            pltpu.VMEM((RCH, SEQ, EMB), jnp.float32),
            pltpu.SemaphoreType.DMA,
        ],
    )(tokens, table)


def kernel(tokens, table):
    return _emb_call(tokens.astype(jnp.int32), table)
